# baseline (speedup 1.0000x reference)
import functools

import jax
import jax.numpy as jnp
from jax import lax
from jax.experimental import pallas as pl
from jax.experimental.pallas import tpu as pltpu
from jax.experimental.pallas import tpu_sc as plsc

DIM = 128
B = 1024
LOUT = 199
NW = 32
G = 64
CPL = B // G
NITEM = LOUT * CPL
IPW = 100
NBUF = 5
NROUND = IPW // NBUF


def _make_gather():
    mesh = plsc.VectorSubcoreMesh(core_axis_name="c", subcore_axis_name="s")

    @functools.partial(
        pl.kernel,
        mesh=mesh,
        out_type=jax.ShapeDtypeStruct((LOUT, B, DIM), jnp.float32),
        compiler_params=pltpu.CompilerParams(use_tc_tiling_on_sc=True),
        scratch_types=[
            pltpu.VMEM((IPW, G), jnp.int32),
        ]
        + [pltpu.VMEM((G, DIM), jnp.float32) for _ in range(NBUF)]
        + [pltpu.SemaphoreType.DMA for _ in range(2 * NBUF)],
    )
    def gather_kernel(idx_hbm, table_hbm, out_hbm, idx_v, *bufs_and_sems):
        bufs = bufs_and_sems[:NBUF]
        gsem = bufs_and_sems[NBUF : 2 * NBUF]
        ssem = bufs_and_sems[2 * NBUF :]
        wid = lax.axis_index("s") * 2 + lax.axis_index("c")
        pltpu.sync_copy(idx_hbm.at[wid], idx_v)

        def out_slab(j):
            g = wid * IPW + j
            item = jnp.where(g < NITEM, g, g - NITEM)
            return out_hbm.at[item >> 4, pl.ds((item & 15) * G, G)]

        def body(t, carry):
            for b in range(NBUF):
                j = t * NBUF + b

                @pl.when(t > 0)
                def _():
                    pltpu.make_async_copy(bufs[b], out_slab(j - NBUF), ssem[b]).wait()

                pltpu.async_copy(table_hbm.at[idx_v.at[j]], bufs[b], gsem[b])
            for b in range(NBUF):
                j = t * NBUF + b
                pltpu.make_async_copy(
                    table_hbm.at[idx_v.at[j]], bufs[b], gsem[b]
                ).wait()
                pltpu.async_copy(bufs[b], out_slab(j), ssem[b])
            return carry

        lax.fori_loop(0, NROUND, body, 0)
        for b in range(NBUF):
            pltpu.make_async_copy(bufs[b], out_slab((NROUND - 1) * NBUF + b), ssem[b]).wait()

    return gather_kernel


_gather = _make_gather()


def kernel(_, decoder_input_ids, probs):
    ids_t = decoder_input_ids[:, 1:].T.reshape(NITEM, G)
    idx = jnp.concatenate([ids_t, ids_t[: NW * IPW - NITEM]]).reshape(NW, IPW, G)
    out_t = _gather(idx, probs)
    return (out_t.transpose(1, 0, 2),)


# speedup vs baseline: 1.0081x; 1.0081x over previous
import functools

import jax
import jax.numpy as jnp
from jax import lax
from jax.experimental import pallas as pl
from jax.experimental.pallas import tpu as pltpu
from jax.experimental.pallas import tpu_sc as plsc

DIM = 128
B = 1024
LOUT = 199
NW = 32
G = 64
CPL = B // G
NITEM = LOUT * CPL
IPW = 100
NBUF = 10
NROUND = IPW // NBUF


def _make_gather():
    mesh = plsc.VectorSubcoreMesh(core_axis_name="c", subcore_axis_name="s")

    @functools.partial(
        pl.kernel,
        mesh=mesh,
        out_type=jax.ShapeDtypeStruct((LOUT, B, DIM), jnp.float32),
        compiler_params=pltpu.CompilerParams(use_tc_tiling_on_sc=True),
        scratch_types=[
            pltpu.VMEM((IPW, G), jnp.int32),
        ]
        + [pltpu.VMEM((G, DIM), jnp.float32) for _ in range(NBUF)]
        + [pltpu.SemaphoreType.DMA for _ in range(2 * NBUF)],
    )
    def gather_kernel(idx_hbm, table_hbm, out_hbm, idx_v, *bufs_and_sems):
        bufs = bufs_and_sems[:NBUF]
        gsem = bufs_and_sems[NBUF : 2 * NBUF]
        ssem = bufs_and_sems[2 * NBUF :]
        wid = lax.axis_index("s") * 2 + lax.axis_index("c")
        pltpu.sync_copy(idx_hbm.at[wid], idx_v)

        def out_slab(j):
            g = wid * IPW + j
            item = jnp.where(g < NITEM, g, g - NITEM)
            return out_hbm.at[item >> 4, pl.ds((item & 15) * G, G)]

        def body(t, carry):
            for b in range(NBUF):
                j = t * NBUF + b

                @pl.when(t > 0)
                def _():
                    pltpu.make_async_copy(bufs[b], out_slab(j - NBUF), ssem[b]).wait()

                pltpu.async_copy(table_hbm.at[idx_v.at[j]], bufs[b], gsem[b])
            for b in range(NBUF):
                j = t * NBUF + b
                pltpu.make_async_copy(
                    table_hbm.at[idx_v.at[j]], bufs[b], gsem[b]
                ).wait()
                pltpu.async_copy(bufs[b], out_slab(j), ssem[b])
            return carry

        lax.fori_loop(0, NROUND, body, 0)
        for b in range(NBUF):
            pltpu.make_async_copy(bufs[b], out_slab((NROUND - 1) * NBUF + b), ssem[b]).wait()

    return gather_kernel


_gather = _make_gather()


def kernel(_, decoder_input_ids, probs):
    ids_t = decoder_input_ids[:, 1:].T.reshape(NITEM, G)
    idx = jnp.concatenate([ids_t, ids_t[: NW * IPW - NITEM]]).reshape(NW, IPW, G)
    out_t = _gather(idx, probs)
    return (out_t.transpose(1, 0, 2),)


# speedup vs baseline: 1.0127x; 1.0046x over previous
import functools

import jax
import jax.numpy as jnp
from jax import lax
from jax.experimental import pallas as pl
from jax.experimental.pallas import tpu as pltpu
from jax.experimental.pallas import tpu_sc as plsc

DIM = 128
B = 1024
LOUT = 199
NW = 32
G = 64
CPL = B // G
NITEM = LOUT * CPL
IPW = 100
NBUF = 10
NROUND = IPW // NBUF


def _make_gather():
    mesh = plsc.VectorSubcoreMesh(core_axis_name="c", subcore_axis_name="s")

    @functools.partial(
        pl.kernel,
        mesh=mesh,
        out_type=jax.ShapeDtypeStruct((LOUT, B, DIM), jnp.float32),
        compiler_params=pltpu.CompilerParams(use_tc_tiling_on_sc=True, skip_device_barrier=True),
        scratch_types=[
            pltpu.VMEM((IPW, G), jnp.int32),
        ]
        + [pltpu.VMEM((G, DIM), jnp.float32) for _ in range(NBUF)]
        + [pltpu.SemaphoreType.DMA for _ in range(2 * NBUF)],
    )
    def gather_kernel(idx_hbm, table_hbm, out_hbm, idx_v, *bufs_and_sems):
        bufs = bufs_and_sems[:NBUF]
        gsem = bufs_and_sems[NBUF : 2 * NBUF]
        ssem = bufs_and_sems[2 * NBUF :]
        wid = lax.axis_index("s") * 2 + lax.axis_index("c")
        pltpu.sync_copy(idx_hbm.at[wid], idx_v)

        def out_slab(j):
            g = wid * IPW + j
            item = jnp.where(g < NITEM, g, g - NITEM)
            return out_hbm.at[item >> 4, pl.ds((item & 15) * G, G)]

        def body(t, carry):
            for b in range(NBUF):
                j = t * NBUF + b

                @pl.when(t > 0)
                def _():
                    pltpu.make_async_copy(bufs[b], out_slab(j - NBUF), ssem[b]).wait()

                pltpu.async_copy(table_hbm.at[idx_v.at[j]], bufs[b], gsem[b])
            for b in range(NBUF):
                j = t * NBUF + b
                pltpu.make_async_copy(
                    table_hbm.at[idx_v.at[j]], bufs[b], gsem[b]
                ).wait()
                pltpu.async_copy(bufs[b], out_slab(j), ssem[b])
            return carry

        lax.fori_loop(0, NROUND, body, 0)
        for b in range(NBUF):
            pltpu.make_async_copy(bufs[b], out_slab((NROUND - 1) * NBUF + b), ssem[b]).wait()

    return gather_kernel


_gather = _make_gather()


def kernel(_, decoder_input_ids, probs):
    ids_t = decoder_input_ids[:, 1:].T.reshape(NITEM, G)
    idx = jnp.concatenate([ids_t, ids_t[: NW * IPW - NITEM]]).reshape(NW, IPW, G)
    out_t = _gather(idx, probs)
    return (out_t.transpose(1, 0, 2),)
